# baseline (speedup 1.0000x reference)
import functools

import jax
import jax.numpy as jnp
from jax import lax
from jax.experimental import pallas as pl
from jax.experimental.pallas import tpu as pltpu
from jax.experimental.pallas import tpu_sc as plsc

_G = 64
_BLK = 1000
_F32 = jnp.float32


def _mm_pool_body(x_ref, w_ref, batch_ref, hl_ref, px_ref):
    i = pl.program_id(0)
    xb = x_ref[...]
    acc = jnp.dot(xb, w_ref[...], preferred_element_type=_F32)
    hl_ref[0] = acc[:, :128]
    hl_ref[1] = acc[:, 128:]
    b = batch_ref[0, 0, :]
    oh = (lax.broadcasted_iota(jnp.int32, (_G, _BLK), 0) == b[None, :]).astype(_F32)

    @pl.when(i == 0)
    def _():
        px_ref[...] = jnp.zeros_like(px_ref)

    px_ref[...] += jnp.dot(oh, xb, preferred_element_type=_F32)


def _stats_body(s_ref, mu_ref, rs_ref, ssum, ssq):
    i = pl.program_id(0)
    nb = pl.num_programs(0)
    sb = s_ref[...]

    @pl.when(i == 0)
    def _():
        ssum[...] = jnp.zeros_like(ssum)
        ssq[...] = jnp.zeros_like(ssq)

    ssum[...] += jnp.sum(sb, axis=1)
    ssq[...] += jnp.sum(sb * sb, axis=1)

    @pl.when(i == nb - 1)
    def _():
        n = jnp.float32(nb * _BLK)
        mu = ssum[...] / n
        var = ssq[...] / n - mu * mu
        mu_ref[...] = mu
        rs_ref[...] = lax.rsqrt(var + 1e-5)


def _bn_mm_pool_body(s_ref, mu_ref, rs_ref, g_ref, be_ref, w1a_ref, w1b_ref,
                     batch_ref, hl_ref, ph_ref):
    i = pl.program_id(0)
    s = s_ref[...]
    scale = (rs_ref[...] * g_ref[...])[:, None, :]
    h = jnp.maximum((s - mu_ref[...][:, None, :]) * scale + be_ref[...][:, None, :], 0.0)
    h0, h1 = h[0], h[1]
    acc = (jnp.dot(h0, w1a_ref[...], preferred_element_type=_F32)
           + jnp.dot(h1, w1b_ref[...], preferred_element_type=_F32))
    hl_ref[0] = acc[:, :128]
    hl_ref[1] = acc[:, 128:]
    b = batch_ref[0, 0, :]
    oh = (lax.broadcasted_iota(jnp.int32, (_G, _BLK), 0) == b[None, :]).astype(_F32)

    @pl.when(i == 0)
    def _():
        ph_ref[...] = jnp.zeros_like(ph_ref)

    ph_ref[:, :128] += jnp.dot(oh, h0, preferred_element_type=_F32)
    ph_ref[:, 128:] += jnp.dot(oh, h1, preferred_element_type=_F32)


def _final_body(s_ref, mu_ref, rs_ref, g_ref, be_ref, batch_ref, px_ref, ph1_ref,
                wp0_ref, wp1_ref, wp2_ref, bps_ref, out_ref, ph2, cnt):
    i = pl.program_id(0)
    nb = pl.num_programs(0)
    s = s_ref[...]
    scale = (rs_ref[...] * g_ref[...])[:, None, :]
    h = jnp.maximum((s - mu_ref[...][:, None, :]) * scale + be_ref[...][:, None, :], 0.0)
    b = batch_ref[0, 0, :]
    oh = (lax.broadcasted_iota(jnp.int32, (_G, _BLK), 0) == b[None, :]).astype(_F32)

    @pl.when(i == 0)
    def _():
        ph2[...] = jnp.zeros_like(ph2)
        cnt[...] = jnp.zeros_like(cnt)

    ph2[:, :128] += jnp.dot(oh, h[0], preferred_element_type=_F32)
    ph2[:, 128:] += jnp.dot(oh, h[1], preferred_element_type=_F32)
    cnt[...] += jnp.broadcast_to(jnp.sum(oh, axis=1, keepdims=True), cnt.shape)

    @pl.when(i == nb - 1)
    def _():
        c = jnp.maximum(cnt[:, 0:1], 1.0)
        r = (jnp.dot(px_ref[...] / c, wp0_ref[...], preferred_element_type=_F32)
             + jnp.dot(ph1_ref[...] / c, wp1_ref[...], preferred_element_type=_F32)
             + jnp.dot(ph2[...] / c, wp2_ref[...], preferred_element_type=_F32)
             + bps_ref[...])
        out_ref[...] = jax.nn.sigmoid(r)


def _make_edge_scatter(n_nodes, n_edges):
    NC, NS = 2, 16
    EPT = n_edges // NS
    K = 112
    NCHUNK = EPT // K
    ZR = 80
    NZCH = n_nodes // ZR
    ZITER = (NZCH + NS - 1) // NS
    mesh = plsc.VectorSubcoreMesh(core_axis_name="c", subcore_axis_name="s")

    @functools.partial(
        pl.kernel,
        out_type=jax.ShapeDtypeStruct((NC, n_nodes, 128), _F32),
        mesh=mesh,
        scratch_types=[
            pltpu.VMEM_SHARED((n_nodes, 128), _F32),
            pltpu.VMEM((3, K), jnp.int32),
            pltpu.VMEM((3, K), jnp.int32),
            pltpu.VMEM((3, K), _F32),
            pltpu.VMEM((3, K), jnp.int32),
            pltpu.VMEM((K, 128), _F32),
            pltpu.VMEM((K, 128), _F32),
            pltpu.VMEM((K, 128), _F32),
            pltpu.SemaphoreType.DMA,
            pltpu.SemaphoreType.DMA,
            pltpu.SemaphoreType.DMA,
            pltpu.SemaphoreType.DMA,
            pltpu.SemaphoreType.DMA,
            pltpu.SemaphoreType.DMA,
            pltpu.SemaphoreType.DMA,
            pltpu.SemaphoreType.DMA,
            pltpu.SemaphoreType.DMA,
        ],
    )
    def edge_scatter(hl_hbm, src_hbm, dst_hbm, w_hbm, out_hbm,
                     acc, src_c, dst_c, w_c, scat, rb0, rb1, rb2,
                     m0, m1, m2, g0, g1, g2, sc0, sc1, sc2):
        cid = lax.axis_index("c")
        sid = lax.axis_index("s")
        table = hl_hbm.at[cid]
        rowsb = (rb0, rb1, rb2)
        rowsf = rowsb
        gsem = (g0, g1, g2)
        msem = (m0, m1, m2)
        ssem = (sc0, sc1, sc2)

        def meta_issue(ci, bi):
            pltpu.async_copy(src_hbm.at[sid].at[ci], src_c.at[bi], msem[bi])
            pltpu.async_copy(dst_hbm.at[sid].at[ci], dst_c.at[bi], msem[bi])
            pltpu.async_copy(w_hbm.at[sid].at[ci], w_c.at[bi], msem[bi])

        def meta_wait(bi):
            pltpu.make_async_copy(src_hbm.at[sid].at[0], src_c.at[bi], msem[bi]).wait()
            pltpu.make_async_copy(src_hbm.at[sid].at[0], dst_c.at[bi], msem[bi]).wait()
            pltpu.make_async_copy(src_hbm.at[sid].at[0], w_c.at[bi], msem[bi]).wait()

        def gather_issue(bi):
            pltpu.async_copy(table.at[src_c.at[bi]], rowsb[bi], gsem[bi])

        def gather_wait(bi):
            pltpu.make_async_copy(table.at[src_c.at[bi]], rowsb[bi], gsem[bi]).wait()

        def scat_issue(bi):
            for g in range(K // 16):
                sl = pl.ds(g * 16, 16)
                scat[bi, sl] = dst_c[bi, sl]
            pltpu.async_copy(rowsf[bi], acc.at[scat.at[bi]], ssem[bi], add=True)

        def scat_wait(bi):
            pltpu.make_async_copy(rowsf[bi], acc.at[scat.at[bi]], ssem[bi]).wait()

        meta_issue(0, 0)
        meta_issue(1, 1)
        meta_issue(2, 2)

        def zb(j, carry):
            for g in range(8):
                rb1[j, pl.ds(g * 16, 16)] = jnp.zeros((16,), _F32)
            return carry

        lax.fori_loop(0, ZR, zb, 0)

        def zc(t, carry):
            ch = sid + t * NS

            @pl.when(ch < NZCH)
            def _():
                pltpu.sync_copy(rb1.at[pl.ds(0, ZR)], acc.at[pl.ds(ch * ZR, ZR)])

            return carry

        lax.fori_loop(0, ZITER, zc, 0)
        plsc.subcore_barrier()

        def scale(bi):
            bsrc = rowsb[bi]
            bdst = rowsf[bi]

            def grp(g, c2):
                w16 = w_c[bi, pl.ds(g * 16, 16)]
                for e in range(16):
                    j = g * 16 + e
                    wb = jnp.full((16,), w16[e], _F32)
                    for c in range(8):
                        sl = pl.ds(c * 16, 16)
                        bdst[j, sl] = bsrc[j, sl] * wb
                return c2

            lax.fori_loop(0, K // 16, grp, 0)

        def step(bi, nmeta):
            gather_wait(bi)
            scale(bi)
            scat_issue(bi)
            meta_issue(nmeta, bi)
            nbi = (bi + 2) % 3
            scat_wait(nbi)
            meta_wait(nbi)
            gather_issue(nbi)

        meta_wait(0)
        gather_issue(0)
        meta_wait(1)
        gather_issue(1)
        gather_wait(0)
        scale(0)
        scat_issue(0)
        meta_issue(3, 0)
        meta_wait(2)
        gather_issue(2)
        step(1, 4)

        def pipe(t, carry):
            c = 2 + 3 * t
            step(2, c + 3)
            step(0, c + 4)
            step(1, c + 5)
            return carry

        lax.fori_loop(0, (NCHUNK - 5) // 3, pipe, 0)
        gather_wait(2)
        scale(2)
        scat_issue(2)
        scat_wait(1)
        meta_wait(1)
        gather_issue(1)
        gather_wait(0)
        scale(0)
        scat_issue(0)
        scat_wait(2)
        gather_wait(1)
        scale(1)
        scat_issue(1)
        scat_wait(0)
        scat_wait(1)
        plsc.subcore_barrier()

        def oc(t, carry):
            ch = sid + t * NS

            @pl.when(ch < NZCH)
            def _():
                r = ch * ZR
                pltpu.sync_copy(acc.at[pl.ds(r, ZR)], out_hbm.at[cid].at[pl.ds(r, ZR)])

            return carry

        lax.fori_loop(0, ZITER, oc, 0)

    return edge_scatter


def kernel(x, edge_index, edge_weight, batch,
           W0, b0, W1, b1, g0, be0, g1, be1,
           Wp0, bp0, Wp1, bp1, Wp2, bp2):
    n, d_in = x.shape
    e = edge_weight.shape[0]
    nb = n // _BLK
    ep = 16 * 92 * 112
    pad = ep - e
    src = jnp.concatenate([edge_index[0], jnp.zeros((pad,), jnp.int32)])
    dst = jnp.concatenate([edge_index[1], jnp.zeros((pad,), jnp.int32)])
    ew = jnp.concatenate([edge_weight, jnp.zeros((pad,), _F32)])
    src = src.reshape(16, 92, 112)
    dst = dst.reshape(16, 92, 112)
    ew = ew.reshape(16, 92, 112)
    batch3 = batch.reshape(nb, 1, _BLK)

    edge_scatter = _make_edge_scatter(n, ep)

    hl0, px = pl.pallas_call(
        _mm_pool_body,
        grid=(nb,),
        in_specs=[
            pl.BlockSpec((_BLK, d_in), lambda i: (i, 0)),
            pl.BlockSpec((d_in, 256), lambda i: (0, 0)),
            pl.BlockSpec((1, 1, _BLK), lambda i: (i, 0, 0)),
        ],
        out_specs=[
            pl.BlockSpec((2, _BLK, 128), lambda i: (0, i, 0)),
            pl.BlockSpec((_G, 256), lambda i: (0, 0)),
        ],
        out_shape=[
            jax.ShapeDtypeStruct((2, n, 128), _F32),
            jax.ShapeDtypeStruct((_G, 256), _F32),
        ],
    )(x, W0, batch3)

    s0 = edge_scatter(hl0, src, dst, ew)

    stats_call = pl.pallas_call(
        _stats_body,
        grid=(nb,),
        in_specs=[pl.BlockSpec((2, _BLK, 128), lambda i: (0, i, 0))],
        out_specs=[
            pl.BlockSpec((2, 128), lambda i: (0, 0)),
            pl.BlockSpec((2, 128), lambda i: (0, 0)),
        ],
        out_shape=[
            jax.ShapeDtypeStruct((2, 128), _F32),
            jax.ShapeDtypeStruct((2, 128), _F32),
        ],
        scratch_shapes=[
            pltpu.VMEM((2, 128), _F32),
            pltpu.VMEM((2, 128), _F32),
        ],
    )
    mu0, rs0 = stats_call(s0)

    hl1, ph1 = pl.pallas_call(
        _bn_mm_pool_body,
        grid=(nb,),
        in_specs=[
            pl.BlockSpec((2, _BLK, 128), lambda i: (0, i, 0)),
            pl.BlockSpec((2, 128), lambda i: (0, 0)),
            pl.BlockSpec((2, 128), lambda i: (0, 0)),
            pl.BlockSpec((2, 128), lambda i: (0, 0)),
            pl.BlockSpec((2, 128), lambda i: (0, 0)),
            pl.BlockSpec((128, 256), lambda i: (0, 0)),
            pl.BlockSpec((128, 256), lambda i: (0, 0)),
            pl.BlockSpec((1, 1, _BLK), lambda i: (i, 0, 0)),
        ],
        out_specs=[
            pl.BlockSpec((2, _BLK, 128), lambda i: (0, i, 0)),
            pl.BlockSpec((_G, 256), lambda i: (0, 0)),
        ],
        out_shape=[
            jax.ShapeDtypeStruct((2, n, 128), _F32),
            jax.ShapeDtypeStruct((_G, 256), _F32),
        ],
    )(s0, mu0, rs0, g0.reshape(2, 128), be0.reshape(2, 128),
      W1[:128], W1[128:], batch3)

    s1 = edge_scatter(hl1, src, dst, ew)

    mu1, rs1 = stats_call(s1)

    out = pl.pallas_call(
        _final_body,
        grid=(nb,),
        in_specs=[
            pl.BlockSpec((2, _BLK, 128), lambda i: (0, i, 0)),
            pl.BlockSpec((2, 128), lambda i: (0, 0)),
            pl.BlockSpec((2, 128), lambda i: (0, 0)),
            pl.BlockSpec((2, 128), lambda i: (0, 0)),
            pl.BlockSpec((2, 128), lambda i: (0, 0)),
            pl.BlockSpec((1, 1, _BLK), lambda i: (i, 0, 0)),
            pl.BlockSpec((_G, 256), lambda i: (0, 0)),
            pl.BlockSpec((_G, 256), lambda i: (0, 0)),
            pl.BlockSpec((256, 128), lambda i: (0, 0)),
            pl.BlockSpec((256, 128), lambda i: (0, 0)),
            pl.BlockSpec((256, 128), lambda i: (0, 0)),
            pl.BlockSpec((1, 128), lambda i: (0, 0)),
        ],
        out_specs=pl.BlockSpec((_G, 128), lambda i: (0, 0)),
        out_shape=jax.ShapeDtypeStruct((_G, 128), _F32),
        scratch_shapes=[
            pltpu.VMEM((_G, 256), _F32),
            pltpu.VMEM((_G, 128), _F32),
        ],
    )(s1, mu1, rs1, g1.reshape(2, 128), be1.reshape(2, 128), batch3,
      px, ph1, Wp0, Wp1, Wp2, (bp0 + bp1 + bp2).reshape(1, 128))

    return out


# speedup vs baseline: 2.6408x; 2.6408x over previous
import functools

import jax
import jax.numpy as jnp
from jax import lax
from jax.experimental import pallas as pl
from jax.experimental.pallas import tpu as pltpu
from jax.experimental.pallas import tpu_sc as plsc

_G = 64
_BLK = 1000
_F32 = jnp.float32


def _mm_pool_body(x_ref, w_ref, batch_ref, hl_ref, px_ref):
    i = pl.program_id(0)
    xb = x_ref[...]
    acc = jnp.dot(xb, w_ref[...], preferred_element_type=_F32)
    hl_ref[0] = acc[:, :128]
    hl_ref[1] = acc[:, 128:]
    b = batch_ref[0, 0, :]
    oh = (lax.broadcasted_iota(jnp.int32, (_G, _BLK), 0) == b[None, :]).astype(_F32)

    @pl.when(i == 0)
    def _():
        px_ref[...] = jnp.zeros_like(px_ref)

    px_ref[...] += jnp.dot(oh, xb, preferred_element_type=_F32)


def _stats_body(s_ref, mu_ref, rs_ref, ssum, ssq):
    i = pl.program_id(0)
    nb = pl.num_programs(0)
    sb = s_ref[...]

    @pl.when(i == 0)
    def _():
        ssum[...] = jnp.zeros_like(ssum)
        ssq[...] = jnp.zeros_like(ssq)

    ssum[...] += jnp.sum(sb, axis=1)
    ssq[...] += jnp.sum(sb * sb, axis=1)

    @pl.when(i == nb - 1)
    def _():
        n = jnp.float32(nb * _BLK)
        mu = ssum[...] / n
        var = ssq[...] / n - mu * mu
        mu_ref[...] = mu
        rs_ref[...] = lax.rsqrt(var + 1e-5)


def _bn_mm_pool_body(s_ref, mu_ref, rs_ref, g_ref, be_ref, w1a_ref, w1b_ref,
                     batch_ref, hl_ref, ph_ref):
    i = pl.program_id(0)
    s = s_ref[...]
    scale = (rs_ref[...] * g_ref[...])[:, None, :]
    h = jnp.maximum((s - mu_ref[...][:, None, :]) * scale + be_ref[...][:, None, :], 0.0)
    h0, h1 = h[0], h[1]
    acc = (jnp.dot(h0, w1a_ref[...], preferred_element_type=_F32)
           + jnp.dot(h1, w1b_ref[...], preferred_element_type=_F32))
    hl_ref[0] = acc[:, :128]
    hl_ref[1] = acc[:, 128:]
    b = batch_ref[0, 0, :]
    oh = (lax.broadcasted_iota(jnp.int32, (_G, _BLK), 0) == b[None, :]).astype(_F32)

    @pl.when(i == 0)
    def _():
        ph_ref[...] = jnp.zeros_like(ph_ref)

    ph_ref[:, :128] += jnp.dot(oh, h0, preferred_element_type=_F32)
    ph_ref[:, 128:] += jnp.dot(oh, h1, preferred_element_type=_F32)


def _final_body(s_ref, mu_ref, rs_ref, g_ref, be_ref, batch_ref, px_ref, ph1_ref,
                wp0_ref, wp1_ref, wp2_ref, bps_ref, out_ref, ph2, cnt):
    i = pl.program_id(0)
    nb = pl.num_programs(0)
    s = s_ref[...]
    scale = (rs_ref[...] * g_ref[...])[:, None, :]
    h = jnp.maximum((s - mu_ref[...][:, None, :]) * scale + be_ref[...][:, None, :], 0.0)
    b = batch_ref[0, 0, :]
    oh = (lax.broadcasted_iota(jnp.int32, (_G, _BLK), 0) == b[None, :]).astype(_F32)

    @pl.when(i == 0)
    def _():
        ph2[...] = jnp.zeros_like(ph2)
        cnt[...] = jnp.zeros_like(cnt)

    ph2[:, :128] += jnp.dot(oh, h[0], preferred_element_type=_F32)
    ph2[:, 128:] += jnp.dot(oh, h[1], preferred_element_type=_F32)
    cnt[...] += jnp.broadcast_to(jnp.sum(oh, axis=1, keepdims=True), cnt.shape)

    @pl.when(i == nb - 1)
    def _():
        c = jnp.maximum(cnt[:, 0:1], 1.0)
        r = (jnp.dot(px_ref[...] / c, wp0_ref[...], preferred_element_type=_F32)
             + jnp.dot(ph1_ref[...] / c, wp1_ref[...], preferred_element_type=_F32)
             + jnp.dot(ph2[...] / c, wp2_ref[...], preferred_element_type=_F32)
             + bps_ref[...])
        out_ref[...] = jax.nn.sigmoid(r)


def _make_edge_scatter(n_nodes, n_edges):
    NC, NS = 2, 16
    EPT = n_edges // NS
    K = 112
    NCHUNK = EPT // K
    ZR = 80
    NZCH = n_nodes // ZR
    ZITER = (NZCH + NS - 1) // NS
    mesh = plsc.VectorSubcoreMesh(core_axis_name="c", subcore_axis_name="s")

    @functools.partial(
        pl.kernel,
        out_type=jax.ShapeDtypeStruct((NC, n_nodes, 128), _F32),
        mesh=mesh,
        scratch_types=[
            pltpu.VMEM_SHARED((n_nodes, 128), _F32),
            pltpu.VMEM((3, K), jnp.int32),
            pltpu.VMEM((3, K), jnp.int32),
            pltpu.VMEM((3, K), _F32),
            pltpu.VMEM((3, K), jnp.int32),
            pltpu.VMEM((K, 128), _F32),
            pltpu.VMEM((K, 128), _F32),
            pltpu.VMEM((K, 128), _F32),
            pltpu.SemaphoreType.DMA,
            pltpu.SemaphoreType.DMA,
            pltpu.SemaphoreType.DMA,
            pltpu.SemaphoreType.DMA,
            pltpu.SemaphoreType.DMA,
            pltpu.SemaphoreType.DMA,
            pltpu.SemaphoreType.DMA,
            pltpu.SemaphoreType.DMA,
            pltpu.SemaphoreType.DMA,
        ],
    )
    def edge_scatter(hl_hbm, src_hbm, dst_hbm, w_hbm, out_hbm,
                     acc, src_c, dst_c, w_c, scat, rb0, rb1, rb2,
                     m0, m1, m2, g0, g1, g2, sc0, sc1, sc2):
        cid = lax.axis_index("c")
        sid = lax.axis_index("s")
        table = hl_hbm.at[cid]
        rowsb = (rb0, rb1, rb2)
        rowsf = rowsb
        gsem = (g0, g1, g2)
        msem = (m0, m1, m2)
        ssem = (sc0, sc1, sc2)

        def meta_issue(ci, bi):
            pltpu.async_copy(src_hbm.at[sid].at[ci], src_c.at[bi], msem[bi])
            pltpu.async_copy(dst_hbm.at[sid].at[ci], dst_c.at[bi], msem[bi])
            pltpu.async_copy(w_hbm.at[sid].at[ci], w_c.at[bi], msem[bi])

        def meta_wait(bi):
            pltpu.make_async_copy(src_hbm.at[sid].at[0], src_c.at[bi], msem[bi]).wait()
            pltpu.make_async_copy(src_hbm.at[sid].at[0], dst_c.at[bi], msem[bi]).wait()
            pltpu.make_async_copy(src_hbm.at[sid].at[0], w_c.at[bi], msem[bi]).wait()

        def gather_issue(bi):
            pltpu.async_copy(table.at[src_c.at[bi]], rowsb[bi], gsem[bi])

        def gather_wait(bi):
            pltpu.make_async_copy(table.at[src_c.at[bi]], rowsb[bi], gsem[bi]).wait()

        def scat_issue(bi):
            for g in range(K // 16):
                sl = pl.ds(g * 16, 16)
                scat[bi, sl] = dst_c[bi, sl]
            pltpu.async_copy(rowsf[bi], acc.at[scat.at[bi]], ssem[bi], add=True)

        def scat_wait(bi):
            pltpu.make_async_copy(rowsf[bi], acc.at[scat.at[bi]], ssem[bi]).wait()

        meta_issue(0, 0)
        meta_issue(1, 1)
        meta_issue(2, 2)

        def zb(j, carry):
            for g in range(8):
                rb1[j, pl.ds(g * 16, 16)] = jnp.zeros((16,), _F32)
            return carry

        lax.fori_loop(0, ZR, zb, 0)

        def zc(t, carry):
            ch = sid + t * NS

            @pl.when(ch < NZCH)
            def _():
                pltpu.sync_copy(rb1.at[pl.ds(0, ZR)], acc.at[pl.ds(ch * ZR, ZR)])

            return carry

        lax.fori_loop(0, ZITER, zc, 0)
        plsc.subcore_barrier()

        def scale(bi):
            bsrc = rowsb[bi]
            bdst = rowsf[bi]

            def grp(g, c2):
                w16 = w_c[bi, pl.ds(g * 16, 16)]
                for e in range(16):
                    j = g * 16 + e
                    wb = jnp.full((16,), w16[e], _F32)
                    for c in range(8):
                        sl = pl.ds(c * 16, 16)
                        bdst[j, sl] = bsrc[j, sl] * wb
                return c2

            lax.fori_loop(0, K // 16, grp, 0)

        def step(bi, nmeta):
            gather_wait(bi)
            scale(bi)
            scat_issue(bi)
            meta_issue(nmeta, bi)
            nbi = (bi + 2) % 3
            scat_wait(nbi)
            meta_wait(nbi)
            gather_issue(nbi)

        meta_wait(0)
        gather_issue(0)
        meta_wait(1)
        gather_issue(1)
        gather_wait(0)
        scale(0)
        scat_issue(0)
        meta_issue(3, 0)
        meta_wait(2)
        gather_issue(2)
        step(1, 4)

        def pipe(t, carry):
            c = 2 + 3 * t
            step(2, c + 3)
            step(0, c + 4)
            step(1, c + 5)
            return carry

        lax.fori_loop(0, (NCHUNK - 5) // 3, pipe, 0)
        gather_wait(2)
        scale(2)
        scat_issue(2)
        scat_wait(1)
        meta_wait(1)
        gather_issue(1)
        gather_wait(0)
        scale(0)
        scat_issue(0)
        scat_wait(2)
        gather_wait(1)
        scale(1)
        scat_issue(1)
        scat_wait(0)
        scat_wait(1)
        plsc.subcore_barrier()

        def oc(t, carry):
            ch = sid + t * NS

            @pl.when(ch < NZCH)
            def _():
                r = ch * ZR
                pltpu.sync_copy(acc.at[pl.ds(r, ZR)], out_hbm.at[cid].at[pl.ds(r, ZR)])

            return carry

        lax.fori_loop(0, ZITER, oc, 0)

    return edge_scatter


def kernel(x, edge_index, edge_weight, batch,
           W0, b0, W1, b1, g0, be0, g1, be1,
           Wp0, bp0, Wp1, bp1, Wp2, bp2):
    n, d_in = x.shape
    e = edge_weight.shape[0]
    nb = n // _BLK
    ep = 16 * 92 * 112
    pad = ep - e
    spread = jnp.arange(pad, dtype=jnp.int32) % n
    src = jnp.concatenate([edge_index[0], spread])
    dst = jnp.concatenate([edge_index[1], spread])
    ew = jnp.concatenate([edge_weight, jnp.zeros((pad,), _F32)])
    src = src.reshape(16, 92, 112)
    dst = dst.reshape(16, 92, 112)
    ew = ew.reshape(16, 92, 112)
    batch3 = batch.reshape(nb, 1, _BLK)

    edge_scatter = _make_edge_scatter(n, ep)

    hl0, px = pl.pallas_call(
        _mm_pool_body,
        grid=(nb,),
        in_specs=[
            pl.BlockSpec((_BLK, d_in), lambda i: (i, 0)),
            pl.BlockSpec((d_in, 256), lambda i: (0, 0)),
            pl.BlockSpec((1, 1, _BLK), lambda i: (i, 0, 0)),
        ],
        out_specs=[
            pl.BlockSpec((2, _BLK, 128), lambda i: (0, i, 0)),
            pl.BlockSpec((_G, 256), lambda i: (0, 0)),
        ],
        out_shape=[
            jax.ShapeDtypeStruct((2, n, 128), _F32),
            jax.ShapeDtypeStruct((_G, 256), _F32),
        ],
    )(x, W0, batch3)

    s0 = edge_scatter(hl0, src, dst, ew)

    stats_call = pl.pallas_call(
        _stats_body,
        grid=(nb,),
        in_specs=[pl.BlockSpec((2, _BLK, 128), lambda i: (0, i, 0))],
        out_specs=[
            pl.BlockSpec((2, 128), lambda i: (0, 0)),
            pl.BlockSpec((2, 128), lambda i: (0, 0)),
        ],
        out_shape=[
            jax.ShapeDtypeStruct((2, 128), _F32),
            jax.ShapeDtypeStruct((2, 128), _F32),
        ],
        scratch_shapes=[
            pltpu.VMEM((2, 128), _F32),
            pltpu.VMEM((2, 128), _F32),
        ],
    )
    mu0, rs0 = stats_call(s0)

    hl1, ph1 = pl.pallas_call(
        _bn_mm_pool_body,
        grid=(nb,),
        in_specs=[
            pl.BlockSpec((2, _BLK, 128), lambda i: (0, i, 0)),
            pl.BlockSpec((2, 128), lambda i: (0, 0)),
            pl.BlockSpec((2, 128), lambda i: (0, 0)),
            pl.BlockSpec((2, 128), lambda i: (0, 0)),
            pl.BlockSpec((2, 128), lambda i: (0, 0)),
            pl.BlockSpec((128, 256), lambda i: (0, 0)),
            pl.BlockSpec((128, 256), lambda i: (0, 0)),
            pl.BlockSpec((1, 1, _BLK), lambda i: (i, 0, 0)),
        ],
        out_specs=[
            pl.BlockSpec((2, _BLK, 128), lambda i: (0, i, 0)),
            pl.BlockSpec((_G, 256), lambda i: (0, 0)),
        ],
        out_shape=[
            jax.ShapeDtypeStruct((2, n, 128), _F32),
            jax.ShapeDtypeStruct((_G, 256), _F32),
        ],
    )(s0, mu0, rs0, g0.reshape(2, 128), be0.reshape(2, 128),
      W1[:128], W1[128:], batch3)

    s1 = edge_scatter(hl1, src, dst, ew)

    mu1, rs1 = stats_call(s1)

    out = pl.pallas_call(
        _final_body,
        grid=(nb,),
        in_specs=[
            pl.BlockSpec((2, _BLK, 128), lambda i: (0, i, 0)),
            pl.BlockSpec((2, 128), lambda i: (0, 0)),
            pl.BlockSpec((2, 128), lambda i: (0, 0)),
            pl.BlockSpec((2, 128), lambda i: (0, 0)),
            pl.BlockSpec((2, 128), lambda i: (0, 0)),
            pl.BlockSpec((1, 1, _BLK), lambda i: (i, 0, 0)),
            pl.BlockSpec((_G, 256), lambda i: (0, 0)),
            pl.BlockSpec((_G, 256), lambda i: (0, 0)),
            pl.BlockSpec((256, 128), lambda i: (0, 0)),
            pl.BlockSpec((256, 128), lambda i: (0, 0)),
            pl.BlockSpec((256, 128), lambda i: (0, 0)),
            pl.BlockSpec((1, 128), lambda i: (0, 0)),
        ],
        out_specs=pl.BlockSpec((_G, 128), lambda i: (0, 0)),
        out_shape=jax.ShapeDtypeStruct((_G, 128), _F32),
        scratch_shapes=[
            pltpu.VMEM((_G, 256), _F32),
            pltpu.VMEM((_G, 128), _F32),
        ],
    )(s1, mu1, rs1, g1.reshape(2, 128), be1.reshape(2, 128), batch3,
      px, ph1, Wp0, Wp1, Wp2, (bp0 + bp1 + bp2).reshape(1, 128))

    return out


# speedup vs baseline: 2.6566x; 1.0060x over previous
import functools

import jax
import jax.numpy as jnp
from jax import lax
from jax.experimental import pallas as pl
from jax.experimental.pallas import tpu as pltpu
from jax.experimental.pallas import tpu_sc as plsc

_G = 64
_BLK = 1000
_F32 = jnp.float32


def _mm_pool_body(x_ref, w_ref, batch_ref, hl_ref, px_ref):
    i = pl.program_id(0)
    xb = x_ref[...]
    acc = jnp.dot(xb, w_ref[...], preferred_element_type=_F32)
    hl_ref[0] = acc[:, :128]
    hl_ref[1] = acc[:, 128:]
    b = batch_ref[0, 0, :]
    oh = (lax.broadcasted_iota(jnp.int32, (_G, _BLK), 0) == b[None, :]).astype(_F32)

    @pl.when(i == 0)
    def _():
        px_ref[...] = jnp.zeros_like(px_ref)

    px_ref[...] += jnp.dot(oh, xb, preferred_element_type=_F32)


def _bn_mm_pool_body(s_ref, g_ref, be_ref, w1a_ref, w1b_ref, batch_ref,
                     hl_ref, ph_ref, ssum, ssq, mu_s, rs_s):
    p = pl.program_id(0)
    i = pl.program_id(1)
    nb = pl.num_programs(1)

    @pl.when((p == 0) & (i == 0))
    def _():
        ssum[...] = jnp.zeros_like(ssum)
        ssq[...] = jnp.zeros_like(ssq)

    @pl.when(p == 0)
    def _():
        sb = s_ref[...]
        ssum[...] += jnp.sum(sb, axis=1)
        ssq[...] += jnp.sum(sb * sb, axis=1)

    @pl.when((p == 0) & (i == nb - 1))
    def _():
        n = jnp.float32(nb * _BLK)
        mu = ssum[...] / n
        var = ssq[...] / n - mu * mu
        mu_s[...] = mu
        rs_s[...] = lax.rsqrt(var + 1e-5)

    @pl.when(p == 1)
    def _():
        s = s_ref[...]
        scale = (rs_s[...] * g_ref[...])[:, None, :]
        h = jnp.maximum((s - mu_s[...][:, None, :]) * scale + be_ref[...][:, None, :], 0.0)
        h0, h1 = h[0], h[1]
        acc = (jnp.dot(h0, w1a_ref[...], preferred_element_type=_F32)
               + jnp.dot(h1, w1b_ref[...], preferred_element_type=_F32))
        hl_ref[0] = acc[:, :128]
        hl_ref[1] = acc[:, 128:]
        b = batch_ref[0, 0, :]
        oh = (lax.broadcasted_iota(jnp.int32, (_G, _BLK), 0) == b[None, :]).astype(_F32)

        @pl.when(i == 0)
        def _():
            ph_ref[...] = jnp.zeros_like(ph_ref)

        ph_ref[:, :128] += jnp.dot(oh, h0, preferred_element_type=_F32)
        ph_ref[:, 128:] += jnp.dot(oh, h1, preferred_element_type=_F32)


def _final_body(s_ref, g_ref, be_ref, batch_ref, px_ref, ph1_ref,
                wp0_ref, wp1_ref, wp2_ref, bps_ref, out_ref,
                ssum, ssq, mu_s, rs_s, ph2, cnt):
    p = pl.program_id(0)
    i = pl.program_id(1)
    nb = pl.num_programs(1)

    @pl.when((p == 0) & (i == 0))
    def _():
        ssum[...] = jnp.zeros_like(ssum)
        ssq[...] = jnp.zeros_like(ssq)

    @pl.when(p == 0)
    def _():
        sb = s_ref[...]
        ssum[...] += jnp.sum(sb, axis=1)
        ssq[...] += jnp.sum(sb * sb, axis=1)

    @pl.when((p == 0) & (i == nb - 1))
    def _():
        n = jnp.float32(nb * _BLK)
        mu = ssum[...] / n
        var = ssq[...] / n - mu * mu
        mu_s[...] = mu
        rs_s[...] = lax.rsqrt(var + 1e-5)

    @pl.when(p == 1)
    def _():
        s = s_ref[...]
        scale = (rs_s[...] * g_ref[...])[:, None, :]
        h = jnp.maximum((s - mu_s[...][:, None, :]) * scale + be_ref[...][:, None, :], 0.0)
        b = batch_ref[0, 0, :]
        oh = (lax.broadcasted_iota(jnp.int32, (_G, _BLK), 0) == b[None, :]).astype(_F32)

        @pl.when(i == 0)
        def _():
            ph2[...] = jnp.zeros_like(ph2)
            cnt[...] = jnp.zeros_like(cnt)

        ph2[:, :128] += jnp.dot(oh, h[0], preferred_element_type=_F32)
        ph2[:, 128:] += jnp.dot(oh, h[1], preferred_element_type=_F32)
        cnt[...] += jnp.broadcast_to(jnp.sum(oh, axis=1, keepdims=True), cnt.shape)

        @pl.when(i == nb - 1)
        def _():
            c = jnp.maximum(cnt[:, 0:1], 1.0)
            r = (jnp.dot(px_ref[...] / c, wp0_ref[...], preferred_element_type=_F32)
                 + jnp.dot(ph1_ref[...] / c, wp1_ref[...], preferred_element_type=_F32)
                 + jnp.dot(ph2[...] / c, wp2_ref[...], preferred_element_type=_F32)
                 + bps_ref[...])
            out_ref[...] = jax.nn.sigmoid(r)


def _make_edge_scatter(n_nodes, n_edges):
    NC, NS = 2, 16
    EPT = n_edges // NS
    K = 112
    NCHUNK = EPT // K
    ZR = 80
    NZCH = n_nodes // ZR
    ZITER = (NZCH + NS - 1) // NS
    mesh = plsc.VectorSubcoreMesh(core_axis_name="c", subcore_axis_name="s")

    @functools.partial(
        pl.kernel,
        out_type=jax.ShapeDtypeStruct((NC, n_nodes, 128), _F32),
        mesh=mesh,
        scratch_types=[
            pltpu.VMEM_SHARED((n_nodes, 128), _F32),
            pltpu.VMEM((3, K), jnp.int32),
            pltpu.VMEM((3, K), jnp.int32),
            pltpu.VMEM((3, K), _F32),
            pltpu.VMEM((3, K), jnp.int32),
            pltpu.VMEM((K, 128), _F32),
            pltpu.VMEM((K, 128), _F32),
            pltpu.VMEM((K, 128), _F32),
            pltpu.SemaphoreType.DMA,
            pltpu.SemaphoreType.DMA,
            pltpu.SemaphoreType.DMA,
            pltpu.SemaphoreType.DMA,
            pltpu.SemaphoreType.DMA,
            pltpu.SemaphoreType.DMA,
            pltpu.SemaphoreType.DMA,
            pltpu.SemaphoreType.DMA,
            pltpu.SemaphoreType.DMA,
        ],
    )
    def edge_scatter(hl_hbm, src_hbm, dst_hbm, w_hbm, out_hbm,
                     acc, src_c, dst_c, w_c, scat, rb0, rb1, rb2,
                     m0, m1, m2, g0, g1, g2, sc0, sc1, sc2):
        cid = lax.axis_index("c")
        sid = lax.axis_index("s")
        table = hl_hbm.at[cid]
        rowsb = (rb0, rb1, rb2)
        rowsf = rowsb
        gsem = (g0, g1, g2)
        msem = (m0, m1, m2)
        ssem = (sc0, sc1, sc2)

        def meta_issue(ci, bi):
            pltpu.async_copy(src_hbm.at[sid].at[ci], src_c.at[bi], msem[bi])
            pltpu.async_copy(dst_hbm.at[sid].at[ci], dst_c.at[bi], msem[bi])
            pltpu.async_copy(w_hbm.at[sid].at[ci], w_c.at[bi], msem[bi])

        def meta_wait(bi):
            pltpu.make_async_copy(src_hbm.at[sid].at[0], src_c.at[bi], msem[bi]).wait()
            pltpu.make_async_copy(src_hbm.at[sid].at[0], dst_c.at[bi], msem[bi]).wait()
            pltpu.make_async_copy(src_hbm.at[sid].at[0], w_c.at[bi], msem[bi]).wait()

        def gather_issue(bi):
            pltpu.async_copy(table.at[src_c.at[bi]], rowsb[bi], gsem[bi])

        def gather_wait(bi):
            pltpu.make_async_copy(table.at[src_c.at[bi]], rowsb[bi], gsem[bi]).wait()

        def scat_issue(bi):
            for g in range(K // 16):
                sl = pl.ds(g * 16, 16)
                scat[bi, sl] = dst_c[bi, sl]
            pltpu.async_copy(rowsf[bi], acc.at[scat.at[bi]], ssem[bi], add=True)

        def scat_wait(bi):
            pltpu.make_async_copy(rowsf[bi], acc.at[scat.at[bi]], ssem[bi]).wait()

        meta_issue(0, 0)
        meta_issue(1, 1)
        meta_issue(2, 2)

        def zb(j, carry):
            for g in range(8):
                rb1[j, pl.ds(g * 16, 16)] = jnp.zeros((16,), _F32)
            return carry

        lax.fori_loop(0, ZR, zb, 0)

        def zc(t, carry):
            ch = sid + t * NS

            @pl.when(ch < NZCH)
            def _():
                pltpu.sync_copy(rb1.at[pl.ds(0, ZR)], acc.at[pl.ds(ch * ZR, ZR)])

            return carry

        lax.fori_loop(0, ZITER, zc, 0)
        plsc.subcore_barrier()

        def scale(bi):
            bsrc = rowsb[bi]
            bdst = rowsf[bi]

            def grp(g, c2):
                w16 = w_c[bi, pl.ds(g * 16, 16)]
                for e in range(16):
                    j = g * 16 + e
                    wb = jnp.full((16,), w16[e], _F32)
                    for c in range(8):
                        sl = pl.ds(c * 16, 16)
                        bdst[j, sl] = bsrc[j, sl] * wb
                return c2

            lax.fori_loop(0, K // 16, grp, 0)

        def step(bi, nmeta):
            gather_wait(bi)
            scale(bi)
            scat_issue(bi)
            meta_issue(nmeta, bi)
            nbi = (bi + 2) % 3
            scat_wait(nbi)
            meta_wait(nbi)
            gather_issue(nbi)

        meta_wait(0)
        gather_issue(0)
        meta_wait(1)
        gather_issue(1)
        gather_wait(0)
        scale(0)
        scat_issue(0)
        meta_issue(3, 0)
        meta_wait(2)
        gather_issue(2)
        step(1, 4)

        def pipe(t, carry):
            c = 2 + 3 * t
            step(2, c + 3)
            step(0, c + 4)
            step(1, c + 5)
            return carry

        lax.fori_loop(0, (NCHUNK - 5) // 3, pipe, 0)
        gather_wait(2)
        scale(2)
        scat_issue(2)
        scat_wait(1)
        meta_wait(1)
        gather_issue(1)
        gather_wait(0)
        scale(0)
        scat_issue(0)
        scat_wait(2)
        gather_wait(1)
        scale(1)
        scat_issue(1)
        scat_wait(0)
        scat_wait(1)
        plsc.subcore_barrier()

        def oc(t, carry):
            ch = sid + t * NS

            @pl.when(ch < NZCH)
            def _():
                r = ch * ZR
                pltpu.sync_copy(acc.at[pl.ds(r, ZR)], out_hbm.at[cid].at[pl.ds(r, ZR)])

            return carry

        lax.fori_loop(0, ZITER, oc, 0)

    return edge_scatter


def kernel(x, edge_index, edge_weight, batch,
           W0, b0, W1, b1, g0, be0, g1, be1,
           Wp0, bp0, Wp1, bp1, Wp2, bp2):
    n, d_in = x.shape
    e = edge_weight.shape[0]
    nb = n // _BLK
    ep = 16 * 92 * 112
    pad = ep - e
    spread = jnp.arange(pad, dtype=jnp.int32) % n
    src = jnp.concatenate([edge_index[0], spread])
    dst = jnp.concatenate([edge_index[1], spread])
    ew = jnp.concatenate([edge_weight, jnp.zeros((pad,), _F32)])
    src = src.reshape(16, 92, 112)
    dst = dst.reshape(16, 92, 112)
    ew = ew.reshape(16, 92, 112)
    batch3 = batch.reshape(nb, 1, _BLK)

    edge_scatter = _make_edge_scatter(n, ep)

    hl0, px = pl.pallas_call(
        _mm_pool_body,
        grid=(nb,),
        in_specs=[
            pl.BlockSpec((_BLK, d_in), lambda i: (i, 0)),
            pl.BlockSpec((d_in, 256), lambda i: (0, 0)),
            pl.BlockSpec((1, 1, _BLK), lambda i: (i, 0, 0)),
        ],
        out_specs=[
            pl.BlockSpec((2, _BLK, 128), lambda i: (0, i, 0)),
            pl.BlockSpec((_G, 256), lambda i: (0, 0)),
        ],
        out_shape=[
            jax.ShapeDtypeStruct((2, n, 128), _F32),
            jax.ShapeDtypeStruct((_G, 256), _F32),
        ],
    )(x, W0, batch3)

    s0 = edge_scatter(hl0, src, dst, ew)

    hl1, ph1 = pl.pallas_call(
        _bn_mm_pool_body,
        grid=(2, nb),
        in_specs=[
            pl.BlockSpec((2, _BLK, 128), lambda p, i: (0, i, 0)),
            pl.BlockSpec((2, 128), lambda p, i: (0, 0)),
            pl.BlockSpec((2, 128), lambda p, i: (0, 0)),
            pl.BlockSpec((128, 256), lambda p, i: (0, 0)),
            pl.BlockSpec((128, 256), lambda p, i: (0, 0)),
            pl.BlockSpec((1, 1, _BLK), lambda p, i: (i, 0, 0)),
        ],
        out_specs=[
            pl.BlockSpec((2, _BLK, 128), lambda p, i: (0, i * p, 0)),
            pl.BlockSpec((_G, 256), lambda p, i: (0, 0)),
        ],
        out_shape=[
            jax.ShapeDtypeStruct((2, n, 128), _F32),
            jax.ShapeDtypeStruct((_G, 256), _F32),
        ],
        scratch_shapes=[
            pltpu.VMEM((2, 128), _F32),
            pltpu.VMEM((2, 128), _F32),
            pltpu.VMEM((2, 128), _F32),
            pltpu.VMEM((2, 128), _F32),
        ],
    )(s0, g0.reshape(2, 128), be0.reshape(2, 128), W1[:128], W1[128:], batch3)

    s1 = edge_scatter(hl1, src, dst, ew)

    out = pl.pallas_call(
        _final_body,
        grid=(2, nb),
        in_specs=[
            pl.BlockSpec((2, _BLK, 128), lambda p, i: (0, i, 0)),
            pl.BlockSpec((2, 128), lambda p, i: (0, 0)),
            pl.BlockSpec((2, 128), lambda p, i: (0, 0)),
            pl.BlockSpec((1, 1, _BLK), lambda p, i: (i, 0, 0)),
            pl.BlockSpec((_G, 256), lambda p, i: (0, 0)),
            pl.BlockSpec((_G, 256), lambda p, i: (0, 0)),
            pl.BlockSpec((256, 128), lambda p, i: (0, 0)),
            pl.BlockSpec((256, 128), lambda p, i: (0, 0)),
            pl.BlockSpec((256, 128), lambda p, i: (0, 0)),
            pl.BlockSpec((1, 128), lambda p, i: (0, 0)),
        ],
        out_specs=pl.BlockSpec((_G, 128), lambda p, i: (0, 0)),
        out_shape=jax.ShapeDtypeStruct((_G, 128), _F32),
        scratch_shapes=[
            pltpu.VMEM((2, 128), _F32),
            pltpu.VMEM((2, 128), _F32),
            pltpu.VMEM((2, 128), _F32),
            pltpu.VMEM((2, 128), _F32),
            pltpu.VMEM((_G, 256), _F32),
            pltpu.VMEM((_G, 128), _F32),
        ],
    )(s1, g1.reshape(2, 128), be1.reshape(2, 128), batch3,
      px, ph1, Wp0, Wp1, Wp2, (bp0 + bp1 + bp2).reshape(1, 128))

    return out
